# Initial kernel scaffold; baseline (speedup 1.0000x reference)
import jax
import jax.numpy as jnp
from jax.experimental import pallas as pl


def kernel(query, key, value, in_proj_weight, in_proj_bias, ln_q_g, ln_q_b, gp_q_w, gp_q_b, ln_k_g, ln_k_b, gp_k_w, gp_k_b, ln_v_g, ln_v_b, gp_v_w, gp_v_b, out_w, out_b):
    raise NotImplementedError("write your pallas kernel here")


# speedup vs baseline: 1.5890x; 1.5890x over previous
import math

import jax
import jax.numpy as jnp
from jax.experimental import pallas as pl

_EMBED = 1024
_HEADS = 16
_HDIM = 64
_SEQ = 2048
_QCHUNK = 512


def _mha_body(xq_ref, xk_ref, xv_ref, wq_ref, wk_ref, wv_ref,
              bq_ref, bk_ref, bv_ref, wo_ref, bo_ref, out_ref):
    h = pl.program_id(0)
    f32 = jnp.float32
    dn = (((1,), (1,)), ((), ()))
    q_h = jax.lax.dot_general(xq_ref[...], wq_ref[...], dn,
                              preferred_element_type=f32) + bq_ref[0]
    k_h = jax.lax.dot_general(xk_ref[...], wk_ref[...], dn,
                              preferred_element_type=f32) + bk_ref[0]
    v_h = jax.lax.dot_general(xv_ref[...], wv_ref[...], dn,
                              preferred_element_type=f32) + bv_ref[0]
    scale = 1.0 / math.sqrt(_HDIM)
    for i in range(_SEQ // _QCHUNK):
        qc = q_h[i * _QCHUNK:(i + 1) * _QCHUNK] * scale
        s = jax.lax.dot_general(qc, k_h, dn, preferred_element_type=f32)
        m = jnp.max(s, axis=-1, keepdims=True)
        e = jnp.exp(s - m)
        p = e / jnp.sum(e, axis=-1, keepdims=True)
        o = jnp.dot(p, v_h, preferred_element_type=f32)
        contrib = jnp.dot(o, wo_ref[...], preferred_element_type=f32)
        sl = pl.ds(i * _QCHUNK, _QCHUNK)

        @pl.when(h == 0)
        def _():
            out_ref[sl, :] = contrib + bo_ref[...]

        @pl.when(h != 0)
        def _():
            out_ref[sl, :] = out_ref[sl, :] + contrib


def kernel(query, key, value, in_proj_weight, in_proj_bias,
           ln_q_g, ln_q_b, gp_q_w, gp_q_b,
           ln_k_g, ln_k_b, gp_k_w, gp_k_b,
           ln_v_g, ln_v_b, gp_v_w, gp_v_b,
           out_w, out_b):
    del ln_q_g, ln_q_b, gp_q_w, gp_q_b, ln_k_g, ln_k_b, gp_k_w, gp_k_b
    del ln_v_g, ln_v_b, gp_v_w, gp_v_b
    xq = query[:, 0, :]
    xk = key[:, 0, :]
    xv = value[:, 0, :]
    b3 = in_proj_bias.reshape(3 * _HEADS, 1, _HDIM)
    bo = out_b.reshape(1, _EMBED)
    out2d = pl.pallas_call(
        _mha_body,
        grid=(_HEADS,),
        in_specs=[
            pl.BlockSpec((_SEQ, _EMBED), lambda h: (0, 0)),
            pl.BlockSpec((_SEQ, _EMBED), lambda h: (0, 0)),
            pl.BlockSpec((_SEQ, _EMBED), lambda h: (0, 0)),
            pl.BlockSpec((_HDIM, _EMBED), lambda h: (h, 0)),
            pl.BlockSpec((_HDIM, _EMBED), lambda h: (_HEADS + h, 0)),
            pl.BlockSpec((_HDIM, _EMBED), lambda h: (2 * _HEADS + h, 0)),
            pl.BlockSpec((1, 1, _HDIM), lambda h: (h, 0, 0)),
            pl.BlockSpec((1, 1, _HDIM), lambda h: (_HEADS + h, 0, 0)),
            pl.BlockSpec((1, 1, _HDIM), lambda h: (2 * _HEADS + h, 0, 0)),
            pl.BlockSpec((_HDIM, _EMBED), lambda h: (h, 0)),
            pl.BlockSpec((1, _EMBED), lambda h: (0, 0)),
        ],
        out_specs=pl.BlockSpec((_SEQ, _EMBED), lambda h: (0, 0)),
        out_shape=jax.ShapeDtypeStruct((_SEQ, _EMBED), jnp.float32),
    )(xq, xk, xv, in_proj_weight, in_proj_weight, in_proj_weight,
      b3, b3, b3, out_w.T, bo)
    return out2d[:, None, :]
